# baseline (speedup 1.0000x reference)
import functools

import jax
import jax.numpy as jnp
from jax import lax
from jax.experimental import pallas as pl
from jax.experimental.pallas import tpu as pltpu
from jax.experimental.pallas import tpu_sc as plsc

_L = 16


@functools.lru_cache(maxsize=None)
def _make_sc_kernel(N, H, V, CH):
    info = plsc.get_sparse_core_info()
    NC, NS = info.num_cores, info.num_subcores
    NW = NC * NS
    assert N % (NW * CH) == 0 and H % _L == 0
    rows_per_w = N // NW
    n_chunks = rows_per_w // CH
    groups = H // _L
    mesh = plsc.VectorSubcoreMesh(core_axis_name="c", subcore_axis_name="s")

    @functools.partial(
        pl.kernel,
        mesh=mesh,
        out_type=jax.ShapeDtypeStruct((N, H), jnp.float32),
        compiler_params=pltpu.CompilerParams(needs_layout_passes=False),
        scratch_types=[
            pltpu.VMEM((V * H,), jnp.float32),
            pltpu.VMEM((CH,), jnp.int32),
            pltpu.VMEM((CH, H), jnp.float32),
        ],
    )
    def k(x_hbm, mask_hbm, table_hbm, out_hbm, tab_v, idx_v, xbuf):
        wid = lax.axis_index("s") * NC + lax.axis_index("c")
        pltpu.sync_copy(table_hbm, tab_v)
        iota = lax.iota(jnp.int32, _L)

        def chunk_body(c, carry):
            base = wid * rows_per_w + c * CH
            pltpu.sync_copy(x_hbm.at[pl.ds(base, CH)], xbuf)
            pltpu.sync_copy(mask_hbm.at[pl.ds(base, CH)], idx_v)

            def row_body(r, carry):
                mvec = plsc.load_gather(idx_v, [jnp.full((_L,), r, jnp.int32)])
                bvec = mvec * H + iota

                def col_body(j, carry):
                    sl = pl.ds(j * _L, _L)
                    t = plsc.load_gather(tab_v, [bvec + j * _L])
                    xbuf[r, sl] = xbuf[r, sl] + t
                    return carry

                return lax.fori_loop(0, groups, col_body, carry)

            lax.fori_loop(0, CH, row_body, 0)
            pltpu.sync_copy(xbuf, out_hbm.at[pl.ds(base, CH)])
            return carry

        lax.fori_loop(0, n_chunks, chunk_body, 0)

    return k


def kernel(input, segment_mask, segment_embedding):
    B, S, H = input.shape
    V = segment_embedding.shape[0]
    N = B * S
    x = input.reshape(N, H)
    m = segment_mask.reshape(N).astype(jnp.int32)
    tab = segment_embedding.reshape(V * H).astype(jnp.float32)
    out = _make_sc_kernel(N, H, V, 64)(x, m, tab)
    return out.reshape(B, S, H)


# speedup vs baseline: 1.4044x; 1.4044x over previous
import functools

import jax
import jax.numpy as jnp
from jax import lax
from jax.experimental import pallas as pl
from jax.experimental.pallas import tpu as pltpu
from jax.experimental.pallas import tpu_sc as plsc

_L = 16


@functools.lru_cache(maxsize=None)
def _make_sc_kernel(N, H, V, CH):
    info = plsc.get_sparse_core_info()
    NC, NS = info.num_cores, info.num_subcores
    NW = NC * NS
    assert N % (NW * CH) == 0 and H % _L == 0
    rows_per_w = N // NW
    n_chunks = rows_per_w // CH
    assert n_chunks % 2 == 0 and n_chunks >= 4
    pairs = n_chunks // 2
    groups = H // _L
    CHH = CH * H
    mesh = plsc.VectorSubcoreMesh(core_axis_name="c", subcore_axis_name="s")

    @functools.partial(
        pl.kernel,
        mesh=mesh,
        out_type=jax.ShapeDtypeStruct((N * H,), jnp.float32),
        compiler_params=pltpu.CompilerParams(needs_layout_passes=False),
        scratch_types=[
            pltpu.VMEM((V * H,), jnp.float32),
            pltpu.VMEM((CH,), jnp.int32),
            pltpu.VMEM((CH,), jnp.int32),
            pltpu.VMEM((CHH,), jnp.float32),
            pltpu.VMEM((CHH,), jnp.float32),
            pltpu.VMEM((CHH,), jnp.float32),
            pltpu.VMEM((CHH,), jnp.float32),
            pltpu.SemaphoreType.DMA,
            pltpu.SemaphoreType.DMA,
            pltpu.SemaphoreType.DMA,
            pltpu.SemaphoreType.DMA,
        ],
    )
    def k(x_hbm, mask_hbm, table_hbm, out_hbm,
          tab_v, idxA, idxB, xA, xB, oA, oB,
          semInA, semInB, semOutA, semOutB):
        wid = lax.axis_index("s") * NC + lax.axis_index("c")
        row0 = wid * rows_per_w
        pltpu.sync_copy(table_hbm, tab_v)
        iota = lax.iota(jnp.int32, _L)

        def in_copy(chunk, xb, idxb, sem):
            rbase = row0 + chunk * CH
            return (
                pltpu.make_async_copy(x_hbm.at[pl.ds(rbase * H, CHH)], xb, sem),
                pltpu.make_async_copy(mask_hbm.at[pl.ds(rbase, CH)], idxb, sem),
            )

        def out_copy(chunk, ob, sem):
            rbase = row0 + chunk * CH
            return pltpu.make_async_copy(ob, out_hbm.at[pl.ds(rbase * H, CHH)], sem)

        def start_in(chunk, xb, idxb, sem):
            for d in in_copy(chunk, xb, idxb, sem):
                d.start()

        def wait_in(chunk, xb, idxb, sem):
            for d in in_copy(chunk, xb, idxb, sem):
                d.wait()

        def compute(xb, ob, idxb):
            @plsc.parallel_loop(0, CH)
            def row_body(r):
                mvec = plsc.load_gather(idxb, [jnp.full((_L,), r, jnp.int32)])
                tbase = jnp.max(mvec) * H
                off = r * H

                @plsc.parallel_loop(0, groups, unroll=8)
                def col_body(j):
                    t = tab_v[pl.ds(tbase + j * _L, _L)]
                    ob[pl.ds(off + j * _L, _L)] = xb[pl.ds(off + j * _L, _L)] + t

        def stage(chunk, xb, idxb, ob, sem_in, sem_out, first):
            wait_in(chunk, xb, idxb, sem_in)
            if not first:
                out_copy(chunk - 2, ob, sem_out).wait()
            compute(xb, ob, idxb)
            out_copy(chunk, ob, sem_out).start()

        start_in(0, xA, idxA, semInA)
        start_in(1, xB, idxB, semInB)
        stage(0, xA, idxA, oA, semInA, semOutA, first=True)
        start_in(2, xA, idxA, semInA)
        stage(1, xB, idxB, oB, semInB, semOutB, first=True)
        start_in(3, xB, idxB, semInB)

        def pair_body(c2, carry):
            c = 2 * c2
            stage(c, xA, idxA, oA, semInA, semOutA, first=False)
            start_in(c + 2, xA, idxA, semInA)
            stage(c + 1, xB, idxB, oB, semInB, semOutB, first=False)
            start_in(c + 3, xB, idxB, semInB)
            return carry

        lax.fori_loop(1, pairs - 1, pair_body, 0)

        c = n_chunks - 2
        stage(c, xA, idxA, oA, semInA, semOutA, first=False)
        stage(c + 1, xB, idxB, oB, semInB, semOutB, first=False)
        out_copy(c, oA, semOutA).wait()
        out_copy(c + 1, oB, semOutB).wait()

    return k


def kernel(input, segment_mask, segment_embedding):
    B, S, H = input.shape
    V = segment_embedding.shape[0]
    N = B * S
    x = input.reshape(N * H)
    m = segment_mask.reshape(N).astype(jnp.int32)
    tab = segment_embedding.reshape(V * H).astype(jnp.float32)
    out = _make_sc_kernel(N, H, V, 16)(x, m, tab)
    return out.reshape(B, S, H)


# speedup vs baseline: 1.4807x; 1.0543x over previous
import functools

import jax
import jax.numpy as jnp
from jax import lax
from jax.experimental import pallas as pl
from jax.experimental.pallas import tpu as pltpu
from jax.experimental.pallas import tpu_sc as plsc

_L = 16
_D = 4


@functools.lru_cache(maxsize=None)
def _make_sc_kernel(N, H, V, CH):
    info = plsc.get_sparse_core_info()
    NC, NS = info.num_cores, info.num_subcores
    NW = NC * NS
    assert N % (NW * CH) == 0 and H % _L == 0
    rows_per_w = N // NW
    n_chunks = rows_per_w // CH
    assert n_chunks % _D == 0 and n_chunks // _D >= 3
    quads = n_chunks // _D
    groups = H // _L
    CHH = CH * H
    mesh = plsc.VectorSubcoreMesh(core_axis_name="c", subcore_axis_name="s")

    @functools.partial(
        pl.kernel,
        mesh=mesh,
        out_type=jax.ShapeDtypeStruct((N * H,), jnp.float32),
        compiler_params=pltpu.CompilerParams(needs_layout_passes=False),
        scratch_types=(
            [pltpu.VMEM((V * H,), jnp.float32)]
            + [pltpu.VMEM((CH,), jnp.int32) for _ in range(_D)]
            + [pltpu.VMEM((CHH,), jnp.float32) for _ in range(_D)]
            + [pltpu.SemaphoreType.DMA for _ in range(2 * _D)]
        ),
    )
    def k(x_hbm, mask_hbm, table_hbm, out_hbm, tab_v, *bufs):
        idxs = bufs[:_D]
        xs = bufs[_D:2 * _D]
        sem_in = bufs[2 * _D:3 * _D]
        sem_out = bufs[3 * _D:4 * _D]
        wid = lax.axis_index("s") * NC + lax.axis_index("c")
        row0 = wid * rows_per_w
        pltpu.sync_copy(table_hbm, tab_v)
        iota = lax.iota(jnp.int32, _L)

        def in_copies(chunk, k):
            rbase = row0 + chunk * CH
            return (
                pltpu.make_async_copy(
                    x_hbm.at[pl.ds(rbase * H, CHH)], xs[k], sem_in[k]),
                pltpu.make_async_copy(
                    mask_hbm.at[pl.ds(rbase, CH)], idxs[k], sem_in[k]),
            )

        def start_in(chunk, k):
            for d in in_copies(chunk, k):
                d.start()

        def wait_in(chunk, k):
            for d in in_copies(chunk, k):
                d.wait()

        def out_copy(chunk, k):
            rbase = row0 + chunk * CH
            return pltpu.make_async_copy(
                xs[k], out_hbm.at[pl.ds(rbase * H, CHH)], sem_out[k])

        def compute(k):
            xb, idxb = xs[k], idxs[k]

            @plsc.parallel_loop(0, CH)
            def row_body(r):
                mvec = plsc.load_gather(idxb, [jnp.full((_L,), r, jnp.int32)])
                bvec = mvec * H + iota
                off = r * H

                @plsc.parallel_loop(0, groups, unroll=8)
                def col_body(j):
                    t = plsc.load_gather(tab_v, [bvec + j * _L])
                    plsc.addupdate(xb.at[pl.ds(off + j * _L, _L)], t)

        for k in range(_D):
            start_in(k, k)

        for k in range(_D):
            wait_in(k, k)
            compute(k)
            out_copy(k, k).start()
            if k >= 1:
                pk = k - 1
                out_copy(pk, pk).wait()
                start_in(pk + _D, pk)

        def quad_body(q, carry):
            c0 = q * _D
            for k in range(_D):
                c = c0 + k
                wait_in(c, k)
                compute(k)
                out_copy(c, k).start()
                pk = (k - 1) % _D
                out_copy(c - 1, pk).wait()
                start_in(c - 1 + _D, pk)
            return carry

        lax.fori_loop(1, quads - 1, quad_body, 0)

        c0 = (quads - 1) * _D
        for k in range(_D):
            c = c0 + k
            wait_in(c, k)
            compute(k)
            out_copy(c, k).start()
            pk = (k - 1) % _D
            out_copy(c - 1, pk).wait()
            if k == 0:
                start_in(c - 1 + _D, pk)
        out_copy(c0 + _D - 1, _D - 1).wait()

    return k


def kernel(input, segment_mask, segment_embedding):
    B, S, H = input.shape
    V = segment_embedding.shape[0]
    N = B * S
    x = input.reshape(N * H)
    m = segment_mask.reshape(N).astype(jnp.int32)
    tab = segment_embedding.reshape(V * H).astype(jnp.float32)
    out = _make_sc_kernel(N, H, V, 16)(x, m, tab)
    return out.reshape(B, S, H)


# speedup vs baseline: 4.3391x; 2.9305x over previous
import functools

import jax
import jax.numpy as jnp
from jax import lax
from jax.experimental import pallas as pl
from jax.experimental.pallas import tpu as pltpu
from jax.experimental.pallas import tpu_sc as plsc

_L = 16
_D = 4


@functools.lru_cache(maxsize=None)
def _make_sc_kernel(N, H, V, CH):
    info = plsc.get_sparse_core_info()
    NC, NS = info.num_cores, info.num_subcores
    NW = NC * NS
    assert N % (NW * CH) == 0 and H % _L == 0
    rows_per_w = N // NW
    n_chunks = rows_per_w // CH
    assert n_chunks % _D == 0 and n_chunks // _D >= 3
    quads = n_chunks // _D
    groups = H // _L
    CHH = CH * H
    mesh = plsc.VectorSubcoreMesh(core_axis_name="c", subcore_axis_name="s")

    @functools.partial(
        pl.kernel,
        mesh=mesh,
        out_type=jax.ShapeDtypeStruct((N, H), jnp.float32),
        compiler_params=pltpu.CompilerParams(needs_layout_passes=False),
        scratch_types=(
            [pltpu.VMEM((V * H,), jnp.float32)]
            + [pltpu.VMEM((CH,), jnp.int32) for _ in range(_D)]
            + [pltpu.VMEM((CH, H), jnp.float32) for _ in range(_D)]
            + [pltpu.SemaphoreType.DMA for _ in range(2 * _D)]
        ),
    )
    def k(x_hbm, mask_hbm, table_hbm, out_hbm, tab_v, *bufs):
        idxs = bufs[:_D]
        xs = bufs[_D:2 * _D]
        sem_in = bufs[2 * _D:3 * _D]
        sem_out = bufs[3 * _D:4 * _D]
        wid = lax.axis_index("s") * NC + lax.axis_index("c")
        row0 = wid * rows_per_w
        pltpu.sync_copy(table_hbm, tab_v)
        iota = lax.iota(jnp.int32, _L)

        def in_copies(chunk, k):
            rbase = row0 + chunk * CH
            return (
                pltpu.make_async_copy(
                    x_hbm.at[pl.ds(rbase, CH)], xs[k], sem_in[k]),
                pltpu.make_async_copy(
                    mask_hbm.at[pl.ds(rbase, CH)], idxs[k], sem_in[k]),
            )

        def start_in(chunk, k):
            for d in in_copies(chunk, k):
                d.start()

        def wait_in(chunk, k):
            for d in in_copies(chunk, k):
                d.wait()

        def out_copy(chunk, k):
            rbase = row0 + chunk * CH
            return pltpu.make_async_copy(
                xs[k], out_hbm.at[pl.ds(rbase, CH)], sem_out[k])

        def compute(k):
            xb, idxb = xs[k], idxs[k]

            @plsc.parallel_loop(0, CH)
            def row_body(r):
                mvec = plsc.load_gather(idxb, [jnp.full((_L,), r, jnp.int32)])
                bvec = mvec * H + iota

                @plsc.parallel_loop(0, groups, unroll=8)
                def col_body(j):
                    t = plsc.load_gather(tab_v, [bvec + j * _L])
                    plsc.addupdate(xb.at[r, pl.ds(j * _L, _L)], t)

        for k in range(_D):
            start_in(k, k)

        for k in range(_D):
            wait_in(k, k)
            compute(k)
            out_copy(k, k).start()
            if k >= 1:
                pk = k - 1
                out_copy(pk, pk).wait()
                start_in(pk + _D, pk)

        def quad_body(q, carry):
            c0 = q * _D
            for k in range(_D):
                c = c0 + k
                wait_in(c, k)
                compute(k)
                out_copy(c, k).start()
                pk = (k - 1) % _D
                out_copy(c - 1, pk).wait()
                start_in(c - 1 + _D, pk)
            return carry

        lax.fori_loop(1, quads - 1, quad_body, 0)

        c0 = (quads - 1) * _D
        for k in range(_D):
            c = c0 + k
            wait_in(c, k)
            compute(k)
            out_copy(c, k).start()
            pk = (k - 1) % _D
            out_copy(c - 1, pk).wait()
            if k == 0:
                start_in(c - 1 + _D, pk)
        out_copy(c0 + _D - 1, _D - 1).wait()

    return k


def kernel(input, segment_mask, segment_embedding):
    B, S, H = input.shape
    V = segment_embedding.shape[0]
    N = B * S
    x = input.reshape(N, H)
    m = segment_mask.reshape(N).astype(jnp.int32)
    tab = segment_embedding.reshape(V * H).astype(jnp.float32)
    out = _make_sc_kernel(N, H, V, 16)(x, m, tab)
    return out.reshape(B, S, H)


# speedup vs baseline: 4.3560x; 1.0039x over previous
import functools

import jax
import jax.numpy as jnp
from jax import lax
from jax.experimental import pallas as pl
from jax.experimental.pallas import tpu as pltpu
from jax.experimental.pallas import tpu_sc as plsc

_L = 16
_D = 4


@functools.lru_cache(maxsize=None)
def _make_sc_kernel(N, H, V, CH):
    info = plsc.get_sparse_core_info()
    NC, NS = info.num_cores, info.num_subcores
    NW = NC * NS
    assert N % (NW * CH) == 0 and H % _L == 0
    rows_per_w = N // NW
    n_chunks = rows_per_w // CH
    assert n_chunks % _D == 0 and n_chunks // _D >= 3
    quads = n_chunks // _D
    groups = H // _L
    CHH = CH * H
    mesh = plsc.VectorSubcoreMesh(core_axis_name="c", subcore_axis_name="s")

    @functools.partial(
        pl.kernel,
        mesh=mesh,
        out_type=jax.ShapeDtypeStruct((N, H), jnp.float32),
        compiler_params=pltpu.CompilerParams(needs_layout_passes=False),
        scratch_types=(
            [pltpu.VMEM((V * H,), jnp.float32)]
            + [pltpu.VMEM((rows_per_w,), jnp.int32)]
            + [pltpu.VMEM((CH, H), jnp.float32) for _ in range(_D)]
            + [pltpu.SemaphoreType.DMA for _ in range(2 * _D)]
        ),
    )
    def k(x_hbm, mask_hbm, table_hbm, out_hbm, tab_v, idx_all, *bufs):
        xs = bufs[:_D]
        sem_in = bufs[_D:2 * _D]
        sem_out = bufs[2 * _D:3 * _D]
        wid = lax.axis_index("s") * NC + lax.axis_index("c")
        row0 = wid * rows_per_w
        iota = lax.iota(jnp.int32, _L)

        def in_copy(chunk, k):
            rbase = row0 + chunk * CH
            return pltpu.make_async_copy(
                x_hbm.at[pl.ds(rbase, CH)], xs[k], sem_in[k])

        def start_in(chunk, k):
            in_copy(chunk, k).start()

        def wait_in(chunk, k):
            in_copy(chunk, k).wait()

        def out_copy(chunk, k):
            rbase = row0 + chunk * CH
            return pltpu.make_async_copy(
                xs[k], out_hbm.at[pl.ds(rbase, CH)], sem_out[k])

        def compute(c, k):
            xb = xs[k]
            crow = c * CH

            @plsc.parallel_loop(0, CH)
            def row_body(r):
                mvec = plsc.load_gather(
                    idx_all, [jnp.full((_L,), crow, jnp.int32) + r])
                bvec = mvec * H + iota

                @plsc.parallel_loop(0, groups, unroll=8)
                def col_body(j):
                    t = plsc.load_gather(tab_v, [bvec + j * _L])
                    plsc.addupdate(xb.at[r, pl.ds(j * _L, _L)], t)

        for k in range(_D):
            start_in(k, k)
        pltpu.sync_copy(table_hbm, tab_v)
        pltpu.sync_copy(mask_hbm.at[pl.ds(row0, rows_per_w)], idx_all)

        for k in range(_D):
            wait_in(k, k)
            compute(k, k)
            out_copy(k, k).start()
            if k >= 1:
                pk = k - 1
                out_copy(pk, pk).wait()
                start_in(pk + _D, pk)

        def quad_body(q, carry):
            c0 = q * _D
            for k in range(_D):
                c = c0 + k
                wait_in(c, k)
                compute(c, k)
                out_copy(c, k).start()
                pk = (k - 1) % _D
                out_copy(c - 1, pk).wait()
                start_in(c - 1 + _D, pk)
            return carry

        lax.fori_loop(1, quads - 1, quad_body, 0)

        c0 = (quads - 1) * _D
        for k in range(_D):
            c = c0 + k
            wait_in(c, k)
            compute(c, k)
            out_copy(c, k).start()
            pk = (k - 1) % _D
            out_copy(c - 1, pk).wait()
            if k == 0:
                start_in(c - 1 + _D, pk)
        out_copy(c0 + _D - 1, _D - 1).wait()

    return k


def kernel(input, segment_mask, segment_embedding):
    B, S, H = input.shape
    V = segment_embedding.shape[0]
    N = B * S
    x = input.reshape(N, H)
    m = segment_mask.reshape(N).astype(jnp.int32)
    tab = segment_embedding.reshape(V * H).astype(jnp.float32)
    out = _make_sc_kernel(N, H, V, 16)(x, m, tab)
    return out.reshape(B, S, H)
